# baseline (speedup 1.0000x reference)
import jax
import jax.numpy as jnp
from jax import lax
from jax.experimental import pallas as pl
from jax.experimental.pallas import tpu as pltpu
from jax.experimental.pallas import tpu_sc as plsc

N_NODES = 50000
EMBED = 256
COMP = 128
T = 8
PER_T = 10000

NC = 2
NS = 16
PER_TILE = PER_T // NS
PER_TILE_PAD = 640
TILE_STRIDE = 624
GROUP = 96
N_STREAMS = 7
N_CHUNKS = 2
CHUNK = 12544
ROWS_PER_TILE = CHUNK // NS
DUMMY_ROW = CHUNK
SC1_BASE = N_NODES - N_CHUNKS * CHUNK


XBLK = 1200
EBLK = 2000


def _pool_mat():
    r = lax.broadcasted_iota(jnp.int32, (EMBED, COMP), 0)
    c = lax.broadcasted_iota(jnp.int32, (EMBED, COMP), 1)
    return jnp.where(r // 2 == c, jnp.float32(0.5), jnp.float32(0.0))


def _pool_block2(x_ref, e_ref, px_ref, pe_ref):
    p = _pool_mat()
    px_ref[...] = lax.dot(x_ref[...], p,
                          precision=lax.Precision.HIGHEST,
                          preferred_element_type=jnp.float32)
    pe_ref[...] = lax.dot(e_ref[...], p,
                          precision=lax.Precision.HIGHEST,
                          preferred_element_type=jnp.float32)


def _pool_block1(x_ref, px_ref):
    px_ref[...] = lax.dot(x_ref[...], _pool_mat(),
                          precision=lax.Precision.HIGHEST,
                          preferred_element_type=jnp.float32)


def _pool_first(x, entire_x, nrows):
    return pl.pallas_call(
        _pool_block2,
        grid=(nrows // XBLK,),
        in_specs=[pl.BlockSpec((XBLK, EMBED), lambda i: (i, 0)),
                  pl.BlockSpec((EBLK, EMBED), lambda i: (i, 0))],
        out_specs=[pl.BlockSpec((XBLK, COMP), lambda i: (i, 0)),
                   pl.BlockSpec((EBLK, COMP), lambda i: (i, 0))],
        out_shape=[jax.ShapeDtypeStruct((nrows, COMP), jnp.float32),
                   jax.ShapeDtypeStruct((entire_x.shape[0], COMP),
                                        jnp.float32)],
    )(x, entire_x)


def _pool_x(x, row0, nrows, blk):
    off = row0 // blk
    return pl.pallas_call(
        _pool_block1,
        grid=(nrows // blk,),
        in_specs=[pl.BlockSpec((blk, EMBED), lambda i: (i + off, 0))],
        out_specs=pl.BlockSpec((blk, COMP), lambda i: (i, 0)),
        out_shape=jax.ShapeDtypeStruct((nrows, COMP), jnp.float32),
    )(x)


def _make_sc_body(t0, t1):
    def _sc_body(px_hbm, pe_hbm, idx_hbm, out_hbm, staging, idx_v, remap,
                 srcpos, table, sem_init, sem_g0, sem_g1, sem_idx):
        c = lax.axis_index("c")
        s = lax.axis_index("s")
        gsems = [sem_g0, sem_g1]

        fill_r = jnp.full((16,), DUMMY_ROW, jnp.int32)
        fill_s = jnp.full((16,), -1, jnp.int32)
        for j in range(N_STREAMS):
            for off in range(0, 128, 16):
                remap[j, pl.ds(off, 16)] = fill_r
                srcpos[j, pl.ds(off, 16)] = fill_s

        def idx_load(t):
            return pltpu.make_async_copy(
                idx_hbm.at[pl.ds(t * PER_T + s * TILE_STRIDE,
                                 PER_TILE_PAD)],
                idx_v, sem_idx)

        idx_load(t0).start()

        def per_t(t, _):
            idx_load(t).wait()
            src0 = (t - t0) * PER_T + s * TILE_STRIDE

            for k in range(N_CHUNKS):
                base = c * SC1_BASE + k * CHUNK

                init = pltpu.async_copy(
                    pe_hbm.at[pl.ds(base + s * ROWS_PER_TILE,
                                    ROWS_PER_TILE)],
                    table.at[pl.ds(s * ROWS_PER_TILE, ROWS_PER_TILE)],
                    sem_init)

                lanes = lax.iota(jnp.int32, 16)
                for i in range(PER_TILE_PAD // 16):
                    v = idx_v[pl.ds(i * 16, 16)]
                    local = v - base
                    slot = i * 16 + lanes
                    ok = ((local >= 0) & (local < CHUNK)
                          & (slot >= s) & (slot < s + PER_TILE))
                    remap[i // 6, pl.ds(i % 6 * 16, 16)] = jnp.where(
                        ok, local, DUMMY_ROW)
                    srcpos[i // 6, pl.ds(i % 6 * 16, 16)] = jnp.where(
                        ok, src0 + i * 16 + lanes, -1)

                gathers = [None] * N_STREAMS
                gathers[0] = pltpu.async_copy(
                    px_hbm.at[plsc.Indices(srcpos.at[0, pl.ds(0, GROUP)],
                                           ignored_value=-1)],
                    staging.at[0], gsems[0])

                init.wait()
                plsc.subcore_barrier()

                for j in range(N_STREAMS):
                    if j + 1 < N_STREAMS:
                        gathers[j + 1] = pltpu.async_copy(
                            px_hbm.at[plsc.Indices(
                                srcpos.at[j + 1, pl.ds(0, GROUP)],
                                ignored_value=-1)],
                            staging.at[(j + 1) % 2], gsems[(j + 1) % 2])
                    gathers[j].wait()
                    pltpu.sync_copy(
                        staging.at[j % 2],
                        table.at[plsc.Indices(remap.at[j, pl.ds(0, GROUP)],
                                              ignored_value=DUMMY_ROW)],
                        add=True)

                if k == N_CHUNKS - 1:
                    idx_load(jnp.minimum(t + 1, t1 - 1)).start()

                plsc.subcore_barrier()

                pltpu.sync_copy(
                    table.at[pl.ds(s * ROWS_PER_TILE, ROWS_PER_TILE)],
                    out_hbm.at[t, pl.ds(base + s * ROWS_PER_TILE,
                                        ROWS_PER_TILE)])
            return 0

        lax.fori_loop(t0, t1, per_t, 0)
        idx_load(t1 - 1).wait()

    return _sc_body


@jax.jit
def kernel(x, entire_x, indices):
    px0, pe = _pool_first(x, entire_x, 30000)

    idx3 = indices.astype(jnp.int32).reshape(-1)

    mesh = plsc.VectorSubcoreMesh(core_axis_name="c", subcore_axis_name="s")
    scratch = [
        pltpu.VMEM((2, GROUP, COMP), jnp.float32),
        pltpu.VMEM((PER_TILE_PAD,), jnp.int32),
        pltpu.VMEM((N_STREAMS, 128), jnp.int32),
        pltpu.VMEM((N_STREAMS, 128), jnp.int32),
        pltpu.VMEM_SHARED((CHUNK, COMP), jnp.float32),
        pltpu.SemaphoreType.DMA,
        pltpu.SemaphoreType.DMA,
        pltpu.SemaphoreType.DMA,
        pltpu.SemaphoreType.DMA,
    ]

    def make_sc(t0, t1, full):
        return pl.kernel(
            _make_sc_body(t0, t1),
            out_type=(jax.ShapeDtypeStruct((T, N_NODES, COMP), jnp.float32)
                      if full else ()),
            mesh=mesh,
            scratch_types=scratch,
        )

    px1 = _pool_x(x, 30000, 30000, 6000)
    px2 = _pool_x(x, 60000, 20000, 4000)
    out0 = make_sc(0, 3, True)(px0, pe, idx3)
    out_ref = jax.new_ref(out0)
    make_sc(3, 6, False)(px1, pe, idx3, out_ref)
    make_sc(6, 8, False)(px2, pe, idx3, out_ref)
    return out_ref[...]


# speedup vs baseline: 1.0082x; 1.0082x over previous
import jax
import jax.numpy as jnp
from jax import lax
from jax.experimental import pallas as pl
from jax.experimental.pallas import tpu as pltpu
from jax.experimental.pallas import tpu_sc as plsc

N_NODES = 50000
EMBED = 256
COMP = 128
T = 8
PER_T = 10000

NC = 2
NS = 16
PER_TILE = PER_T // NS
PER_TILE_PAD = 640
TILE_STRIDE = 624
GROUP = 96
N_STREAMS = 7
N_CHUNKS = 2
CHUNK = 12544
ROWS_PER_TILE = CHUNK // NS
DUMMY_ROW = CHUNK
SC1_BASE = N_NODES - N_CHUNKS * CHUNK


HALF_X = T // 2 * PER_T
XBLK = 4000
EBLK = 5000
XBLK2 = 8000


def _pool_mat():
    r = lax.broadcasted_iota(jnp.int32, (EMBED, COMP), 0)
    c = lax.broadcasted_iota(jnp.int32, (EMBED, COMP), 1)
    return jnp.where(r // 2 == c, jnp.float32(0.5), jnp.float32(0.0))


def _pool_block2(x_ref, e_ref, px_ref, pe_ref):
    p = _pool_mat()
    px_ref[...] = lax.dot(x_ref[...], p,
                          precision=lax.Precision.HIGHEST,
                          preferred_element_type=jnp.float32)
    pe_ref[...] = lax.dot(e_ref[...], p,
                          precision=lax.Precision.HIGHEST,
                          preferred_element_type=jnp.float32)


def _pool_block1(x_ref, px_ref):
    px_ref[...] = lax.dot(x_ref[...], _pool_mat(),
                          precision=lax.Precision.HIGHEST,
                          preferred_element_type=jnp.float32)


def _pool_first(x, entire_x):
    return pl.pallas_call(
        _pool_block2,
        grid=(HALF_X // XBLK,),
        in_specs=[pl.BlockSpec((XBLK, EMBED), lambda i: (i, 0)),
                  pl.BlockSpec((EBLK, EMBED), lambda i: (i, 0))],
        out_specs=[pl.BlockSpec((XBLK, COMP), lambda i: (i, 0)),
                   pl.BlockSpec((EBLK, COMP), lambda i: (i, 0))],
        out_shape=[jax.ShapeDtypeStruct((HALF_X, COMP), jnp.float32),
                   jax.ShapeDtypeStruct((entire_x.shape[0], COMP),
                                        jnp.float32)],
    )(x, entire_x)


def _pool_second(x):
    nblk = HALF_X // XBLK2
    return pl.pallas_call(
        _pool_block1,
        grid=(nblk,),
        in_specs=[pl.BlockSpec((XBLK2, EMBED), lambda i: (i + nblk, 0))],
        out_specs=pl.BlockSpec((XBLK2, COMP), lambda i: (i, 0)),
        out_shape=jax.ShapeDtypeStruct((HALF_X, COMP), jnp.float32),
    )(x)


def _make_sc_body(t0, t1):
    def _sc_body(px_hbm, pe_hbm, idx_hbm, out_hbm, staging, idx_v, remap,
                 srcpos, table, sem_init, sem_g0, sem_g1, sem_idx):
        c = lax.axis_index("c")
        s = lax.axis_index("s")
        gsems = [sem_g0, sem_g1]

        fill_r = jnp.full((16,), DUMMY_ROW, jnp.int32)
        fill_s = jnp.full((16,), -1, jnp.int32)
        for j in range(N_STREAMS):
            for off in range(0, 128, 16):
                remap[j, pl.ds(off, 16)] = fill_r
                srcpos[j, pl.ds(off, 16)] = fill_s

        def idx_load(t):
            return pltpu.make_async_copy(
                idx_hbm.at[pl.ds(t * PER_T + s * TILE_STRIDE,
                                 PER_TILE_PAD)],
                idx_v, sem_idx)

        idx_load(t0).start()

        def per_t(t, _):
            idx_load(t).wait()
            src0 = (t - t0) * PER_T + s * TILE_STRIDE

            for k in range(N_CHUNKS):
                base = c * SC1_BASE + k * CHUNK

                init = pltpu.async_copy(
                    pe_hbm.at[pl.ds(base + s * ROWS_PER_TILE,
                                    ROWS_PER_TILE)],
                    table.at[pl.ds(s * ROWS_PER_TILE, ROWS_PER_TILE)],
                    sem_init)

                lanes = lax.iota(jnp.int32, 16)
                for i in range(PER_TILE_PAD // 16):
                    v = idx_v[pl.ds(i * 16, 16)]
                    local = v - base
                    slot = i * 16 + lanes
                    ok = ((local >= 0) & (local < CHUNK)
                          & (slot >= s) & (slot < s + PER_TILE))
                    remap[i // 6, pl.ds(i % 6 * 16, 16)] = jnp.where(
                        ok, local, DUMMY_ROW)
                    srcpos[i // 6, pl.ds(i % 6 * 16, 16)] = jnp.where(
                        ok, src0 + i * 16 + lanes, -1)

                gathers = [None] * N_STREAMS
                gathers[0] = pltpu.async_copy(
                    px_hbm.at[plsc.Indices(srcpos.at[0, pl.ds(0, GROUP)],
                                           ignored_value=-1)],
                    staging.at[0], gsems[0])

                init.wait()
                plsc.subcore_barrier()

                for j in range(N_STREAMS):
                    if j + 1 < N_STREAMS:
                        gathers[j + 1] = pltpu.async_copy(
                            px_hbm.at[plsc.Indices(
                                srcpos.at[j + 1, pl.ds(0, GROUP)],
                                ignored_value=-1)],
                            staging.at[(j + 1) % 2], gsems[(j + 1) % 2])
                    gathers[j].wait()
                    pltpu.sync_copy(
                        staging.at[j % 2],
                        table.at[plsc.Indices(remap.at[j, pl.ds(0, GROUP)],
                                              ignored_value=DUMMY_ROW)],
                        add=True)

                if k == N_CHUNKS - 1:
                    idx_load(jnp.minimum(t + 1, t1 - 1)).start()

                plsc.subcore_barrier()

                pltpu.sync_copy(
                    table.at[pl.ds(s * ROWS_PER_TILE, ROWS_PER_TILE)],
                    out_hbm.at[t, pl.ds(base + s * ROWS_PER_TILE,
                                        ROWS_PER_TILE)])
            return 0

        lax.fori_loop(t0, t1, per_t, 0)
        idx_load(t1 - 1).wait()

    return _sc_body


@jax.jit
def kernel(x, entire_x, indices):
    px0, pe = _pool_first(x, entire_x)

    idx3 = indices.astype(jnp.int32).reshape(-1)

    mesh = plsc.VectorSubcoreMesh(core_axis_name="c", subcore_axis_name="s")
    scratch = [
        pltpu.VMEM((2, GROUP, COMP), jnp.float32),
        pltpu.VMEM((PER_TILE_PAD,), jnp.int32),
        pltpu.VMEM((N_STREAMS, 128), jnp.int32),
        pltpu.VMEM((N_STREAMS, 128), jnp.int32),
        pltpu.VMEM_SHARED((CHUNK, COMP), jnp.float32),
        pltpu.SemaphoreType.DMA,
        pltpu.SemaphoreType.DMA,
        pltpu.SemaphoreType.DMA,
        pltpu.SemaphoreType.DMA,
    ]

    sc1 = pl.kernel(
        _make_sc_body(0, T // 2),
        out_type=jax.ShapeDtypeStruct((T, N_NODES, COMP), jnp.float32),
        mesh=mesh,
        scratch_types=scratch,
    )
    px1 = _pool_second(x)
    out0 = sc1(px0, pe, idx3)

    out_ref = jax.new_ref(out0)
    sc2 = pl.kernel(
        _make_sc_body(T // 2, T),
        out_type=(),
        mesh=mesh,
        scratch_types=scratch,
    )
    sc2(px1, pe, idx3, out_ref)
    return out_ref[...]


# speedup vs baseline: 1.0150x; 1.0067x over previous
import jax
import jax.numpy as jnp
from jax import lax
from jax.experimental import pallas as pl
from jax.experimental.pallas import tpu as pltpu
from jax.experimental.pallas import tpu_sc as plsc

N_NODES = 50000
EMBED = 256
COMP = 128
T = 8
PER_T = 10000

NC = 2
NS = 16
PER_TILE = PER_T // NS
PER_TILE_PAD = 640
TILE_STRIDE = 624
GROUP = 96
N_STREAMS = 7
N_CHUNKS = 2
CHUNK = 12544
ROWS_PER_TILE = CHUNK // NS
DUMMY_ROW = CHUNK
SC1_BASE = N_NODES - N_CHUNKS * CHUNK


HALF_X = T // 2 * PER_T
XBLK = 4000
EBLK = 5000
XBLK2 = 8000


def _pool_mat():
    r = lax.broadcasted_iota(jnp.int32, (EMBED, COMP), 0)
    c = lax.broadcasted_iota(jnp.int32, (EMBED, COMP), 1)
    return jnp.where(r // 2 == c, jnp.float32(0.5), jnp.float32(0.0))


def _pool_block2(x_ref, e_ref, px_ref, pe_ref):
    p = _pool_mat()
    px_ref[...] = lax.dot(x_ref[...], p,
                          precision=lax.Precision.HIGHEST,
                          preferred_element_type=jnp.float32)
    pe_ref[...] = lax.dot(e_ref[...], p,
                          precision=lax.Precision.HIGHEST,
                          preferred_element_type=jnp.float32)


def _pool_block1(x_ref, px_ref):
    px_ref[...] = lax.dot(x_ref[...], _pool_mat(),
                          precision=lax.Precision.HIGHEST,
                          preferred_element_type=jnp.float32)


def _pool_first(x, entire_x):
    return pl.pallas_call(
        _pool_block2,
        grid=(HALF_X // XBLK,),
        in_specs=[pl.BlockSpec((XBLK, EMBED), lambda i: (i, 0)),
                  pl.BlockSpec((EBLK, EMBED), lambda i: (i, 0))],
        out_specs=[pl.BlockSpec((XBLK, COMP), lambda i: (i, 0)),
                   pl.BlockSpec((EBLK, COMP), lambda i: (i, 0))],
        out_shape=[jax.ShapeDtypeStruct((HALF_X, COMP), jnp.float32),
                   jax.ShapeDtypeStruct((entire_x.shape[0], COMP),
                                        jnp.float32)],
    )(x, entire_x)


def _pool_second(x):
    nblk = HALF_X // XBLK2
    return pl.pallas_call(
        _pool_block1,
        grid=(nblk,),
        in_specs=[pl.BlockSpec((XBLK2, EMBED), lambda i: (i + nblk, 0))],
        out_specs=pl.BlockSpec((XBLK2, COMP), lambda i: (i, 0)),
        out_shape=jax.ShapeDtypeStruct((HALF_X, COMP), jnp.float32),
    )(x)


def _make_sc_body(t0, t1):
    def _sc_body(px_hbm, pe_hbm, idx_hbm, out_hbm, staging, idx_v, remap,
                 srcpos, table, sem_init, sem_g0, sem_g1, sem_idx):
        c = lax.axis_index("c")
        s = lax.axis_index("s")
        gsems = [sem_g0, sem_g1]

        fill_r = jnp.full((16,), DUMMY_ROW, jnp.int32)
        fill_s = jnp.full((16,), -1, jnp.int32)
        for j in range(N_STREAMS):
            for off in range(0, 128, 16):
                remap[j, pl.ds(off, 16)] = fill_r
                srcpos[j, pl.ds(off, 16)] = fill_s

        def idx_load(t):
            return pltpu.make_async_copy(
                idx_hbm.at[pl.ds(t * PER_T + s * TILE_STRIDE,
                                 PER_TILE_PAD)],
                idx_v, sem_idx)

        idx_load(t0).start()

        def per_t(t, _):
            idx_load(t).wait()
            src0 = (t - t0) * PER_T + s * TILE_STRIDE

            for k in range(N_CHUNKS):
                base = c * SC1_BASE + k * CHUNK

                init = pltpu.async_copy(
                    pe_hbm.at[pl.ds(base + s * ROWS_PER_TILE,
                                    ROWS_PER_TILE)],
                    table.at[pl.ds(s * ROWS_PER_TILE, ROWS_PER_TILE)],
                    sem_init)

                lanes = lax.iota(jnp.int32, 16)
                for i in range(PER_TILE_PAD // 16):
                    v = idx_v[pl.ds(i * 16, 16)]
                    local = v - base
                    slot = i * 16 + lanes
                    ok = ((local >= 0) & (local < CHUNK)
                          & (slot >= s) & (slot < s + PER_TILE))
                    remap[i // 6, pl.ds(i % 6 * 16, 16)] = jnp.where(
                        ok, local, DUMMY_ROW)
                    srcpos[i // 6, pl.ds(i % 6 * 16, 16)] = jnp.where(
                        ok, src0 + i * 16 + lanes, -1)

                gathers = [None] * N_STREAMS
                for j in range(2):
                    gathers[j] = pltpu.async_copy(
                        px_hbm.at[plsc.Indices(srcpos.at[j, pl.ds(0, GROUP)],
                                               ignored_value=-1)],
                        staging.at[j], gsems[j])

                init.wait()
                plsc.subcore_barrier()

                for j in range(N_STREAMS):
                    if j + 2 < N_STREAMS:
                        gathers[j + 2] = None
                    gathers[j].wait()
                    pltpu.sync_copy(
                        staging.at[j % 2],
                        table.at[plsc.Indices(remap.at[j, pl.ds(0, GROUP)],
                                              ignored_value=DUMMY_ROW)],
                        add=True)
                    if j + 2 < N_STREAMS:
                        gathers[j + 2] = pltpu.async_copy(
                            px_hbm.at[plsc.Indices(
                                srcpos.at[j + 2, pl.ds(0, GROUP)],
                                ignored_value=-1)],
                            staging.at[j % 2], gsems[j % 2])

                if k == N_CHUNKS - 1:
                    idx_load(jnp.minimum(t + 1, t1 - 1)).start()

                plsc.subcore_barrier()

                pltpu.sync_copy(
                    table.at[pl.ds(s * ROWS_PER_TILE, ROWS_PER_TILE)],
                    out_hbm.at[t, pl.ds(base + s * ROWS_PER_TILE,
                                        ROWS_PER_TILE)])
            return 0

        lax.fori_loop(t0, t1, per_t, 0)
        idx_load(t1 - 1).wait()

    return _sc_body


@jax.jit
def kernel(x, entire_x, indices):
    px0, pe = _pool_first(x, entire_x)

    idx3 = indices.astype(jnp.int32).reshape(-1)

    mesh = plsc.VectorSubcoreMesh(core_axis_name="c", subcore_axis_name="s")
    scratch = [
        pltpu.VMEM((2, GROUP, COMP), jnp.float32),
        pltpu.VMEM((PER_TILE_PAD,), jnp.int32),
        pltpu.VMEM((N_STREAMS, 128), jnp.int32),
        pltpu.VMEM((N_STREAMS, 128), jnp.int32),
        pltpu.VMEM_SHARED((CHUNK, COMP), jnp.float32),
        pltpu.SemaphoreType.DMA,
        pltpu.SemaphoreType.DMA,
        pltpu.SemaphoreType.DMA,
        pltpu.SemaphoreType.DMA,
    ]

    sc1 = pl.kernel(
        _make_sc_body(0, T // 2),
        out_type=jax.ShapeDtypeStruct((T, N_NODES, COMP), jnp.float32),
        mesh=mesh,
        scratch_types=scratch,
    )
    px1 = _pool_second(x)
    out0 = sc1(px0, pe, idx3)

    out_ref = jax.new_ref(out0)
    sc2 = pl.kernel(
        _make_sc_body(T // 2, T),
        out_type=(),
        mesh=mesh,
        scratch_types=scratch,
    )
    sc2(px1, pe, idx3, out_ref)
    return out_ref[...]
